# Initial kernel scaffold; baseline (speedup 1.0000x reference)
import jax
import jax.numpy as jnp
from jax.experimental import pallas as pl


def kernel(obj_vecs, pred_vecs, edges, net1_w1, net1_b1, net1_w2, net1_b2, net2_w1, net2_b1, net2_w2, net2_b2):
    raise NotImplementedError("write your pallas kernel here")


# speedup vs baseline: 3.0298x; 3.0298x over previous
import functools

import jax
import jax.numpy as jnp
from jax import lax
from jax.experimental import pallas as pl
from jax.experimental.pallas import tpu as pltpu
from jax.experimental.pallas import tpu_sc as plsc

F32 = jnp.float32
NC, NS = 2, 16
NW = NC * NS
CH = 128
ZR = 80


def _sc_mesh():
    return plsc.VectorSubcoreMesh(core_axis_name="c", subcore_axis_name="s",
                                  num_cores=NC, num_subcores=NS)


def _gather_call(obj_vecs, s_idx, o_idx):
    O, D = obj_vecs.shape
    T = s_idx.shape[0]
    n_chunks = T // CH
    k_iter = (n_chunks + NW - 1) // NW

    @functools.partial(
        pl.kernel,
        out_type=(jax.ShapeDtypeStruct((T, D), F32),
                  jax.ShapeDtypeStruct((T, D), F32)),
        mesh=_sc_mesh(),
        scratch_types=[
            pltpu.VMEM((CH,), jnp.int32),
            pltpu.VMEM((CH,), jnp.int32),
            pltpu.VMEM((CH, D), F32),
            pltpu.VMEM((CH, D), F32),
            pltpu.SemaphoreType.DMA,
            pltpu.SemaphoreType.DMA,
        ],
    )
    def gather_kernel(obj_hbm, sidx_hbm, oidx_hbm, curs_hbm, curo_hbm,
                      sidx_v, oidx_v, srows_v, orows_v, sem_s, sem_o):
        wid = lax.axis_index("s") * NC + lax.axis_index("c")

        def body(k, carry):
            chunk = k * NW + wid

            @pl.when(chunk < n_chunks)
            def _():
                base = chunk * CH
                pltpu.sync_copy(sidx_hbm.at[pl.ds(base, CH)], sidx_v)
                pltpu.sync_copy(oidx_hbm.at[pl.ds(base, CH)], oidx_v)
                cs = pltpu.async_copy(obj_hbm.at[sidx_v], srows_v, sem_s)
                co = pltpu.async_copy(obj_hbm.at[oidx_v], orows_v, sem_o)
                cs.wait()
                pltpu.sync_copy(srows_v, curs_hbm.at[pl.ds(base, CH)])
                co.wait()
                pltpu.sync_copy(orows_v, curo_hbm.at[pl.ds(base, CH)])
            return carry

        lax.fori_loop(0, k_iter, body, 0)

    return gather_kernel(obj_vecs, s_idx, o_idx)


def _scatter_call(new_s, new_o, s_idx, o_idx, O):
    T, D = new_s.shape
    n_chunks = T // CH
    per_sc = n_chunks // NC
    k_iter = (per_sc + NS - 1) // NS
    n_zch = O // ZR
    z_iter = (n_zch + NS - 1) // NS

    @functools.partial(
        pl.kernel,
        out_type=jax.ShapeDtypeStruct((NC, O, D), F32),
        mesh=_sc_mesh(),
        scratch_types=[
            pltpu.VMEM((CH,), jnp.int32),
            pltpu.VMEM((CH, D), F32),
            pltpu.VMEM((ZR, D), F32),
            pltpu.VMEM_SHARED((O, D), F32),
        ],
    )
    def scatter_kernel(news_hbm, newo_hbm, sidx_hbm, oidx_hbm, out_hbm,
                       idx_v, rows_v, zero_v, acc_sh):
        cid = lax.axis_index("c")
        sid = lax.axis_index("s")

        def zbody(r, carry):
            for c in range(D // 16):
                zero_v[r, pl.ds(c * 16, 16)] = jnp.zeros((16,), F32)
            return carry

        lax.fori_loop(0, ZR, zbody, 0)

        def zcopy(k, carry):
            zc = k * NS + sid

            @pl.when(zc < n_zch)
            def _():
                pltpu.sync_copy(zero_v, acc_sh.at[pl.ds(zc * ZR, ZR)])
            return carry

        lax.fori_loop(0, z_iter, zcopy, 0)
        plsc.subcore_barrier()

        def body(k, carry):
            rel = k * NS + sid

            @pl.when(rel < per_sc)
            def _():
                base = (cid * per_sc + rel) * CH
                pltpu.sync_copy(sidx_hbm.at[pl.ds(base, CH)], idx_v)
                pltpu.sync_copy(news_hbm.at[pl.ds(base, CH)], rows_v)
                pltpu.sync_copy(rows_v, acc_sh.at[idx_v], add=True)
                pltpu.sync_copy(oidx_hbm.at[pl.ds(base, CH)], idx_v)
                pltpu.sync_copy(newo_hbm.at[pl.ds(base, CH)], rows_v)
                pltpu.sync_copy(rows_v, acc_sh.at[idx_v], add=True)
            return carry

        lax.fori_loop(0, k_iter, body, 0)
        plsc.subcore_barrier()

        def wcopy(k, carry):
            zc = k * NS + sid

            @pl.when(zc < n_zch)
            def _():
                pltpu.sync_copy(acc_sh.at[pl.ds(zc * ZR, ZR)],
                                out_hbm.at[cid, pl.ds(zc * ZR, ZR)])
            return carry

        lax.fori_loop(0, z_iter, wcopy, 0)

    return scatter_kernel(new_s, new_o, s_idx, o_idx)


def _counts_call(s_idx, o_idx, O, D):
    T = s_idx.shape[0]
    n_chunks = T // CH
    per_sc = n_chunks // NC
    k_iter = (per_sc + NS - 1) // NS
    n_zch = O // ZR
    z_iter = (n_zch + NS - 1) // NS

    @functools.partial(
        pl.kernel,
        out_type=jax.ShapeDtypeStruct((NC, O, D), F32),
        mesh=_sc_mesh(),
        scratch_types=[
            pltpu.VMEM((CH,), jnp.int32),
            pltpu.VMEM((CH, D), F32),
            pltpu.VMEM((ZR, D), F32),
            pltpu.VMEM_SHARED((O, D), F32),
        ],
    )
    def counts_kernel(sidx_hbm, oidx_hbm, out_hbm,
                      idx_v, ones_v, zero_v, acc_sh):
        cid = lax.axis_index("c")
        sid = lax.axis_index("s")

        def fbody(r, carry):
            for c in range(D // 16):
                ones_v[r, pl.ds(c * 16, 16)] = jnp.ones((16,), F32)
            return carry

        lax.fori_loop(0, CH, fbody, 0)

        def zbody(r, carry):
            for c in range(D // 16):
                zero_v[r, pl.ds(c * 16, 16)] = jnp.zeros((16,), F32)
            return carry

        lax.fori_loop(0, ZR, zbody, 0)

        def zcopy(k, carry):
            zc = k * NS + sid

            @pl.when(zc < n_zch)
            def _():
                pltpu.sync_copy(zero_v, acc_sh.at[pl.ds(zc * ZR, ZR)])
            return carry

        lax.fori_loop(0, z_iter, zcopy, 0)
        plsc.subcore_barrier()

        def body(k, carry):
            rel = k * NS + sid

            @pl.when(rel < per_sc)
            def _():
                base = (cid * per_sc + rel) * CH
                pltpu.sync_copy(sidx_hbm.at[pl.ds(base, CH)], idx_v)
                pltpu.sync_copy(ones_v, acc_sh.at[idx_v], add=True)
                pltpu.sync_copy(oidx_hbm.at[pl.ds(base, CH)], idx_v)
                pltpu.sync_copy(ones_v, acc_sh.at[idx_v], add=True)
            return carry

        lax.fori_loop(0, k_iter, body, 0)
        plsc.subcore_barrier()

        def wcopy(k, carry):
            zc = k * NS + sid

            @pl.when(zc < n_zch)
            def _():
                pltpu.sync_copy(acc_sh.at[pl.ds(zc * ZR, ZR)],
                                out_hbm.at[cid, pl.ds(zc * ZR, ZR)])
            return carry

        lax.fori_loop(0, z_iter, wcopy, 0)

    return counts_kernel(s_idx, o_idx)


def _edge_mlp_call(cur_s, pred, cur_o, w1, b1, w2, b2, H, D, block=1000):
    T = cur_s.shape[0]
    grid = T // block

    def body(s_ref, p_ref, o_ref, w1_ref, b1_ref, w2_ref, b2_ref,
             ns_ref, np_ref, no_ref):
        h = (jnp.dot(s_ref[...], w1_ref[0], preferred_element_type=F32) +
             jnp.dot(p_ref[...], w1_ref[1], preferred_element_type=F32) +
             jnp.dot(o_ref[...], w1_ref[2], preferred_element_type=F32))
        h = jnp.maximum(h + b1_ref[...], 0.0)
        t = jnp.dot(h, w2_ref[...], preferred_element_type=F32) + b2_ref[...]
        t = jnp.maximum(t, 0.0)
        ns_ref[...] = t[:, :H]
        np_ref[...] = t[:, H:H + D]
        no_ref[...] = t[:, H + D:]

    row_spec = pl.BlockSpec((block, D), lambda i: (i, 0))
    return pl.pallas_call(
        body,
        grid=(grid,),
        in_specs=[
            row_spec, row_spec, row_spec,
            pl.BlockSpec((3, D, H), lambda i: (0, 0, 0)),
            pl.BlockSpec((1, H), lambda i: (0, 0)),
            pl.BlockSpec((H, 2 * H + D), lambda i: (0, 0)),
            pl.BlockSpec((1, 2 * H + D), lambda i: (0, 0)),
        ],
        out_specs=[pl.BlockSpec((block, H), lambda i: (i, 0)),
                   pl.BlockSpec((block, D), lambda i: (i, 0)),
                   pl.BlockSpec((block, H), lambda i: (i, 0))],
        out_shape=[jax.ShapeDtypeStruct((T, H), F32),
                   jax.ShapeDtypeStruct((T, D), F32),
                   jax.ShapeDtypeStruct((T, H), F32)],
    )(cur_s, pred, cur_o, w1, b1, w2, b2)


def _node_mlp_call(pool_parts, cnt_parts, w1, b1, w2, b2, block=2000):
    _, O, D = pool_parts.shape
    H = w1.shape[1]
    grid = O // block

    def body(pp_ref, cc_ref, w1_ref, b1_ref, w2_ref, b2_ref, out_ref):
        pooled = pp_ref[0] + pp_ref[1]
        cnt = jnp.maximum(cc_ref[0] + cc_ref[1], 1.0)
        pooled = pooled / cnt
        h = jnp.maximum(
            jnp.dot(pooled, w1_ref[...], preferred_element_type=F32)
            + b1_ref[...], 0.0)
        out_ref[...] = jnp.maximum(
            jnp.dot(h, w2_ref[...], preferred_element_type=F32)
            + b2_ref[...], 0.0)

    part_spec = pl.BlockSpec((NC, block, D), lambda i: (0, i, 0))
    return pl.pallas_call(
        body,
        grid=(grid,),
        in_specs=[
            part_spec, part_spec,
            pl.BlockSpec((D, H), lambda i: (0, 0)),
            pl.BlockSpec((1, H), lambda i: (0, 0)),
            pl.BlockSpec((H, D), lambda i: (0, 0)),
            pl.BlockSpec((1, D), lambda i: (0, 0)),
        ],
        out_specs=pl.BlockSpec((block, D), lambda i: (i, 0)),
        out_shape=jax.ShapeDtypeStruct((O, D), F32),
    )(pool_parts, cnt_parts, w1, b1, w2, b2)


def kernel(obj_vecs, pred_vecs, edges, net1_w1, net1_b1, net1_w2, net1_b2,
           net2_w1, net2_b1, net2_w2, net2_b2):
    O, D = obj_vecs.shape
    T = pred_vecs.shape[0]
    L = net1_w1.shape[0]
    H = net1_w1.shape[2]
    s_idx = edges[:, 0]
    o_idx = edges[:, 1]

    cnt_parts = _counts_call(s_idx, o_idx, O, D)

    w1r = net1_w1.reshape(L, 3, D, H)
    b1r = net1_b1.reshape(L, 1, H)
    b2r = net1_b2.reshape(L, 1, 2 * H + D)
    n2b1 = net2_b1.reshape(L, 1, H)
    n2b2 = net2_b2.reshape(L, 1, D)

    for l in range(L):
        cur_s, cur_o = _gather_call(obj_vecs, s_idx, o_idx)
        new_s, new_p, new_o = _edge_mlp_call(
            cur_s, pred_vecs, cur_o, w1r[l], b1r[l], net1_w2[l], b2r[l], H, D)
        pool_parts = _scatter_call(new_s, new_o, s_idx, o_idx, O)
        obj_vecs = _node_mlp_call(
            pool_parts, cnt_parts, net2_w1[l], n2b1[l], net2_w2[l], n2b2[l])
        pred_vecs = new_p
    return (obj_vecs, pred_vecs)


# speedup vs baseline: 4.2677x; 1.4086x over previous
import functools

import jax
import jax.numpy as jnp
from jax import lax
from jax.experimental import pallas as pl
from jax.experimental.pallas import tpu as pltpu
from jax.experimental.pallas import tpu_sc as plsc

F32 = jnp.float32
I32 = jnp.int32
NC, NS = 2, 16
NW = NC * NS
CH = 128
ZR = 40
RING = 4
LOOK = 2
SRING = 2
SLOOK = 1


def _sc_mesh():
    return plsc.VectorSubcoreMesh(core_axis_name="c", subcore_axis_name="s",
                                  num_cores=NC, num_subcores=NS)


GRING = 2


def _gather_call(tabs, idx_g, T_pad):
    _, O, D = tabs.shape
    n_chunks = T_pad // CH
    nch_w = n_chunks // NS
    n_zch = O // ZR
    z_iter = (n_zch + NS - 1) // NS

    @functools.partial(
        pl.kernel,
        out_type=jax.ShapeDtypeStruct((2, T_pad, D), F32),
        mesh=_sc_mesh(),
        scratch_types=[
            pltpu.VMEM((CH,), I32),
            pltpu.VMEM((CH,), I32),
            pltpu.VMEM((GRING, CH, D), F32),
            pltpu.VMEM_SHARED((O, D), F32),
            pltpu.SemaphoreType.DMA,
            pltpu.SemaphoreType.DMA,
            pltpu.SemaphoreType.DMA,
            pltpu.SemaphoreType.DMA,
            pltpu.SemaphoreType.DMA,
            pltpu.SemaphoreType.DMA,
        ],
    )
    def gather_kernel(tabs_hbm, idx_hbm, out_hbm,
                      i0, i1, rows_v, tab_sh,
                      s0, s1, g0, g1, w0, w1):
        cid = lax.axis_index("c")
        sid = lax.axis_index("s")
        idxb = (i0, i1)
        isem = (s0, s1)
        gsem = (g0, g1)
        wsem = (w0, w1)
        my0 = sid * nch_w

        def stage(k, carry):
            zc = k * NS + sid

            @pl.when(zc < n_zch)
            def _():
                pltpu.sync_copy(tabs_hbm.at[cid, pl.ds(zc * ZR, ZR)],
                                tab_sh.at[pl.ds(zc * ZR, ZR)])
            return carry

        lax.fori_loop(0, z_iter, stage, 0)
        plsc.subcore_barrier()

        def i_issue(k, p):
            pltpu.async_copy(idx_hbm.at[cid, my0 + k], idxb[p], isem[p])

        def i_wait(k, p):
            pltpu.make_async_copy(idx_hbm.at[cid, my0 + k], idxb[p],
                                  isem[p]).wait()

        def g_issue(k, p):
            pltpu.async_copy(tab_sh.at[idxb[p]], rows_v.at[p], gsem[p])

        def g_wait(k, p):
            pltpu.make_async_copy(tab_sh.at[idxb[p]], rows_v.at[p],
                                  gsem[p]).wait()

        def w_issue(k, p):
            pltpu.async_copy(rows_v.at[p],
                             out_hbm.at[cid, pl.ds((my0 + k) * CH, CH)],
                             wsem[p])

        def w_wait(k, p):
            pltpu.make_async_copy(rows_v.at[p],
                                  out_hbm.at[cid, pl.ds((my0 + k) * CH, CH)],
                                  wsem[p]).wait()

        i_issue(0, 0)
        i_issue(1, 1)
        i_wait(0, 0)
        g_issue(0, 0)

        def step(g, carry):
            for j in range(GRING):
                k = g * GRING + j
                pn = (j + 1) % GRING

                @pl.when(k + 1 < nch_w)
                def _(k=k, pn=pn):
                    @pl.when(k >= 1)
                    def _():
                        w_wait(k - 1, pn)
                    i_wait(k + 1, pn)
                    g_issue(k + 1, pn)

                g_wait(k, j)
                w_issue(k, j)

                @pl.when(k + 2 < nch_w)
                def _(k=k, j=j):
                    i_issue(k + 2, j)
            return carry

        lax.fori_loop(0, nch_w // GRING, step, 0)
        for k in range(nch_w - GRING, nch_w):
            w_wait(k, k % GRING)

    return gather_kernel(tabs, idx_g)


def _scatter_call(new_s, new_o, sidx_s, oidx_s, O):
    T_pad, D = new_s.shape
    n_chunks = T_pad // CH
    per_sc = n_chunks // NC
    nch_w = per_sc // NS
    n_zch = O // ZR
    z_iter = (n_zch + NS - 1) // NS

    @functools.partial(
        pl.kernel,
        out_type=jax.ShapeDtypeStruct((NC, O, D), F32),
        mesh=_sc_mesh(),
        scratch_types=[
            pltpu.VMEM((CH,), I32),
            pltpu.VMEM((CH,), I32),
            pltpu.VMEM((CH,), I32),
            pltpu.VMEM((CH,), I32),
            pltpu.VMEM((SRING, CH, D), F32),
            pltpu.VMEM((ZR, D), F32),
            pltpu.VMEM_SHARED((O + 8, D), F32),
            pltpu.SemaphoreType.DMA,
            pltpu.SemaphoreType.DMA,
            pltpu.SemaphoreType.DMA,
            pltpu.SemaphoreType.DMA,
        ],
    )
    def scatter_kernel(news_hbm, newo_hbm, sidx_hbm, oidx_hbm, out_hbm,
                       si0, si1, oi0, oi1, rows_v, zero_v, acc_sh,
                       l0, l1, a0, a1):
        cid = lax.axis_index("c")
        sid = lax.axis_index("s")
        lsem = (l0, l1)
        asem = (a0, a1)
        sidx_b = (si0, si1)
        oidx_b = (oi0, oi1)

        def zbody(r, carry):
            for c in range(D // 16):
                zero_v[r, pl.ds(c * 16, 16)] = jnp.zeros((16,), F32)
            return carry

        lax.fori_loop(0, ZR, zbody, 0)

        def zcopy(k, carry):
            zc = k * NS + sid

            @pl.when(zc < n_zch)
            def _():
                pltpu.sync_copy(zero_v, acc_sh.at[pl.ds(zc * ZR, ZR)])
            return carry

        lax.fori_loop(0, z_iter, zcopy, 0)
        plsc.subcore_barrier()

        my0 = cid * per_sc + sid * nch_w

        for side in range(2):
            vals = (news_hbm, newo_hbm)[side]
            idx_hbm = (sidx_hbm, oidx_hbm)[side]
            idx_b = (sidx_b, oidx_b)[side]

            def l_issue(k, p, vals=vals, idx_hbm=idx_hbm, idx_b=idx_b):
                pltpu.async_copy(idx_hbm.at[my0 + k], idx_b[p], lsem[p])
                pltpu.async_copy(vals.at[pl.ds((my0 + k) * CH, CH)],
                                 rows_v.at[p], lsem[p])

            def l_wait(k, p, vals=vals, idx_hbm=idx_hbm, idx_b=idx_b):
                pltpu.make_async_copy(idx_hbm.at[my0 + k], idx_b[p],
                                      lsem[p]).wait()
                pltpu.make_async_copy(vals.at[pl.ds((my0 + k) * CH, CH)],
                                      rows_v.at[p], lsem[p]).wait()

            def a_issue(k, p, idx_b=idx_b):
                pltpu.async_copy(rows_v.at[p], acc_sh.at[idx_b[p]],
                                 asem[p], add=True)

            def a_wait(k, p, idx_b=idx_b):
                pltpu.make_async_copy(rows_v.at[p], acc_sh.at[idx_b[p]],
                                      asem[p]).wait()

            l_issue(0, 0)

            def step(g, carry):
                for j in range(SRING):
                    k = g * SRING + j
                    l_wait(k, j)
                    a_issue(k, j)
                    pnext = (j + SLOOK) % SRING

                    @pl.when(k + SLOOK < nch_w)
                    def _(k=k, j=j, pnext=pnext):
                        @pl.when(k >= SLOOK)
                        def _():
                            a_wait(k - SLOOK, pnext)
                        l_issue(k + SLOOK, pnext)
                return carry

            lax.fori_loop(0, nch_w // SRING, step, 0)
            for k in range(nch_w - SRING, nch_w):
                a_wait(k, k % SRING)

        plsc.subcore_barrier()

        def wcopy(k, carry):
            zc = k * NS + sid

            @pl.when(zc < n_zch)
            def _():
                pltpu.sync_copy(acc_sh.at[pl.ds(zc * ZR, ZR)],
                                out_hbm.at[cid, pl.ds(zc * ZR, ZR)])
            return carry

        lax.fori_loop(0, z_iter, wcopy, 0)

    return scatter_kernel(new_s, new_o, sidx_s, oidx_s)


def _counts_call(s_idx, o_idx, O, D):
    T = s_idx.shape[0]
    n_chunks = T // CH
    per_sc = n_chunks // NC
    k_iter = (per_sc + NS - 1) // NS
    n_zch = O // ZR
    z_iter = (n_zch + NS - 1) // NS

    @functools.partial(
        pl.kernel,
        out_type=jax.ShapeDtypeStruct((NC, O, D), F32),
        mesh=_sc_mesh(),
        scratch_types=[
            pltpu.VMEM((CH,), I32),
            pltpu.VMEM((CH, D), F32),
            pltpu.VMEM((ZR, D), F32),
            pltpu.VMEM_SHARED((O, D), F32),
        ],
    )
    def counts_kernel(sidx_hbm, oidx_hbm, out_hbm,
                      idx_v, ones_v, zero_v, acc_sh):
        cid = lax.axis_index("c")
        sid = lax.axis_index("s")

        def fbody(r, carry):
            for c in range(D // 16):
                ones_v[r, pl.ds(c * 16, 16)] = jnp.ones((16,), F32)
            return carry

        lax.fori_loop(0, CH, fbody, 0)

        def zbody(r, carry):
            for c in range(D // 16):
                zero_v[r, pl.ds(c * 16, 16)] = jnp.zeros((16,), F32)
            return carry

        lax.fori_loop(0, ZR, zbody, 0)

        def zcopy(k, carry):
            zc = k * NS + sid

            @pl.when(zc < n_zch)
            def _():
                pltpu.sync_copy(zero_v, acc_sh.at[pl.ds(zc * ZR, ZR)])
            return carry

        lax.fori_loop(0, z_iter, zcopy, 0)
        plsc.subcore_barrier()

        def body(k, carry):
            rel = k * NS + sid

            @pl.when(rel < per_sc)
            def _():
                base = (cid * per_sc + rel) * CH
                pltpu.sync_copy(sidx_hbm.at[pl.ds(base, CH)], idx_v)
                pltpu.sync_copy(ones_v, acc_sh.at[idx_v], add=True)
                pltpu.sync_copy(oidx_hbm.at[pl.ds(base, CH)], idx_v)
                pltpu.sync_copy(ones_v, acc_sh.at[idx_v], add=True)
            return carry

        lax.fori_loop(0, k_iter, body, 0)
        plsc.subcore_barrier()

        def wcopy(k, carry):
            zc = k * NS + sid

            @pl.when(zc < n_zch)
            def _():
                pltpu.sync_copy(acc_sh.at[pl.ds(zc * ZR, ZR)],
                                out_hbm.at[cid, pl.ds(zc * ZR, ZR)])
            return carry

        lax.fori_loop(0, z_iter, wcopy, 0)

    return counts_kernel(s_idx, o_idx)


def _edge_mlp_call(gso, pred, w1p, b1, w2, b2, H, D, block=1000):
    _, T_pad, _ = gso.shape
    T = pred.shape[0]
    grid = T // block

    def body(gs_ref, go_ref, p_ref, w1_ref, b1_ref, w2_ref, b2_ref,
             ns_ref, np_ref, no_ref):
        h = gs_ref[0] + go_ref[0] + jnp.dot(
            p_ref[...], w1_ref[...], preferred_element_type=F32)
        h = jnp.maximum(h + b1_ref[...], 0.0)
        t = jnp.dot(h, w2_ref[...], preferred_element_type=F32) + b2_ref[...]
        t = jnp.maximum(t, 0.0)
        ns_ref[...] = t[:, :H]
        np_ref[...] = t[:, H:H + D]
        no_ref[...] = t[:, H + D:]

    return pl.pallas_call(
        body,
        grid=(grid,),
        in_specs=[
            pl.BlockSpec((1, block, D), lambda i: (0, i, 0)),
            pl.BlockSpec((1, block, D), lambda i: (1, i, 0)),
            pl.BlockSpec((block, D), lambda i: (i, 0)),
            pl.BlockSpec((D, H), lambda i: (0, 0)),
            pl.BlockSpec((1, H), lambda i: (0, 0)),
            pl.BlockSpec((H, 2 * H + D), lambda i: (0, 0)),
            pl.BlockSpec((1, 2 * H + D), lambda i: (0, 0)),
        ],
        out_specs=[pl.BlockSpec((block, H), lambda i: (i, 0)),
                   pl.BlockSpec((block, D), lambda i: (i, 0)),
                   pl.BlockSpec((block, H), lambda i: (i, 0))],
        out_shape=[jax.ShapeDtypeStruct((T_pad, H), F32),
                   jax.ShapeDtypeStruct((T, D), F32),
                   jax.ShapeDtypeStruct((T_pad, H), F32)],
    )(gso, gso, pred, w1p, b1, w2, b2)


def _node_mlp_call(pool_parts, cnt_parts, w1, b1, w2, b2,
                   w_next_s, w_next_o, block=2000):
    _, O, D = pool_parts.shape
    H = w1.shape[1]
    grid = O // block

    def body(pp_ref, cc_ref, w1_ref, b1_ref, w2_ref, b2_ref,
             wns_ref, wno_ref, out_ref, tabs_ref):
        cnt = jnp.maximum(cc_ref[0] + cc_ref[1], 1.0)
        pooled = (pp_ref[0] + pp_ref[1]) / cnt
        h = jnp.maximum(
            jnp.dot(pooled, w1_ref[...], preferred_element_type=F32)
            + b1_ref[...], 0.0)
        obj = jnp.maximum(
            jnp.dot(h, w2_ref[...], preferred_element_type=F32)
            + b2_ref[...], 0.0)
        out_ref[...] = obj
        tabs_ref[0] = jnp.dot(obj, wns_ref[...], preferred_element_type=F32)
        tabs_ref[1] = jnp.dot(obj, wno_ref[...], preferred_element_type=F32)

    part_spec = pl.BlockSpec((NC, block, D), lambda i: (0, i, 0))
    w_spec = pl.BlockSpec((D, H), lambda i: (0, 0))
    return pl.pallas_call(
        body,
        grid=(grid,),
        in_specs=[
            part_spec, part_spec,
            pl.BlockSpec((D, H), lambda i: (0, 0)),
            pl.BlockSpec((1, H), lambda i: (0, 0)),
            pl.BlockSpec((H, D), lambda i: (0, 0)),
            pl.BlockSpec((1, D), lambda i: (0, 0)),
            w_spec, w_spec,
        ],
        out_specs=[pl.BlockSpec((block, D), lambda i: (i, 0)),
                   pl.BlockSpec((2, block, H), lambda i: (0, i, 0))],
        out_shape=[jax.ShapeDtypeStruct((O, D), F32),
                   jax.ShapeDtypeStruct((2, O, H), F32)],
    )(pool_parts, cnt_parts, w1, b1, w2, b2, w_next_s, w_next_o)


def _contrib_call(obj_vecs, w_s, w_o, block=2000):
    O, D = obj_vecs.shape
    H = w_s.shape[1]
    grid = O // block

    def body(x_ref, ws_ref, wo_ref, tabs_ref):
        tabs_ref[0] = jnp.dot(x_ref[...], ws_ref[...],
                              preferred_element_type=F32)
        tabs_ref[1] = jnp.dot(x_ref[...], wo_ref[...],
                              preferred_element_type=F32)

    return pl.pallas_call(
        body,
        grid=(grid,),
        in_specs=[pl.BlockSpec((block, D), lambda i: (i, 0)),
                  pl.BlockSpec((D, H), lambda i: (0, 0)),
                  pl.BlockSpec((D, H), lambda i: (0, 0))],
        out_specs=pl.BlockSpec((2, block, H), lambda i: (0, i, 0)),
        out_shape=jax.ShapeDtypeStruct((2, O, H), F32),
    )(obj_vecs, w_s, w_o)


def kernel(obj_vecs, pred_vecs, edges, net1_w1, net1_b1, net1_w2, net1_b2,
           net2_w1, net2_b1, net2_w2, net2_b2):
    O, D = obj_vecs.shape
    T = pred_vecs.shape[0]
    L = net1_w1.shape[0]
    H = net1_w1.shape[2]
    s_idx = edges[:, 0]
    o_idx = edges[:, 1]

    n_chunks = T // CH
    n_chunks_pad = ((n_chunks + NW - 1) // NW) * NW
    n_pad = n_chunks_pad - n_chunks
    T_pad = n_chunks_pad * CH
    sidx2 = s_idx.reshape(n_chunks, CH)
    oidx2 = o_idx.reshape(n_chunks, CH)
    pad_g = jnp.zeros((n_pad, CH), I32)
    pad_s = jnp.full((n_pad, CH), O, I32)
    idx_g = jnp.stack([jnp.concatenate([sidx2, pad_g], axis=0),
                       jnp.concatenate([oidx2, pad_g], axis=0)])
    sidx_s = jnp.concatenate([sidx2, pad_s], axis=0)
    oidx_s = jnp.concatenate([oidx2, pad_s], axis=0)

    cnt_parts = _counts_call(s_idx, o_idx, O, D)

    w1r = net1_w1.reshape(L, 3, D, H)
    b1r = net1_b1.reshape(L, 1, H)
    b2r = net1_b2.reshape(L, 1, 2 * H + D)
    n2b1 = net2_b1.reshape(L, 1, H)
    n2b2 = net2_b2.reshape(L, 1, D)

    tabs = _contrib_call(obj_vecs, w1r[0, 0], w1r[0, 2])

    for l in range(L):
        gso = _gather_call(tabs, idx_g, T_pad)
        new_s, new_p, new_o = _edge_mlp_call(
            gso, pred_vecs, w1r[l, 1], b1r[l], net1_w2[l], b2r[l], H, D)
        pool_parts = _scatter_call(new_s, new_o, sidx_s, oidx_s, O)
        ln = min(l + 1, L - 1)
        obj_vecs, tabs = _node_mlp_call(
            pool_parts, cnt_parts, net2_w1[l], n2b1[l], net2_w2[l], n2b2[l],
            w1r[ln, 0], w1r[ln, 2])
        pred_vecs = new_p
    return (obj_vecs, pred_vecs)
